# baseline (speedup 1.0000x reference)
import functools

import jax
import jax.numpy as jnp
from jax import lax
from jax.experimental import pallas as pl
from jax.experimental.pallas import tpu as pltpu
from jax.experimental.pallas import tpu_sc as plsc

N = 10000
E = 320000
F = 128

NC = 2
NS = 16
NW = NC * NS
CH = 80
NCHUNKS = E // CH
NBUF = 4
ZROWS = 32
BASE_CK = NCHUNKS // NW
EXTRA_TILES = NCHUNKS - NW * BASE_CK
CK_HI = -(-(BASE_CK + 1) // NBUF) * NBUF
N_ACC = 10112
ROWS_PER_SUB = N_ACC // NS
ROWS_LAST = N - (NS - 1) * ROWS_PER_SUB


def _sc_segment_sum(feat, edge_flat):
    mesh = plsc.VectorSubcoreMesh(core_axis_name="c", subcore_axis_name="s")

    @functools.partial(
        pl.kernel,
        mesh=mesh,
        out_type=jax.ShapeDtypeStruct((NC, N, F), jnp.float32),
        scratch_types=[
            pltpu.VMEM((NBUF, CH), jnp.int32),
            pltpu.VMEM((NBUF, CH), jnp.int32),
            pltpu.VMEM((NBUF, CH, F), jnp.float32),
            pltpu.VMEM((ZROWS, F), jnp.float32),
            pltpu.VMEM_SHARED((N_ACC, F), jnp.float32),
        ] + [pltpu.SemaphoreType.DMA] * (4 * NBUF + 1),
    )
    def k(feat_hbm, edge_hbm, out_hbm,
          sidx_v, didx_v, rows_v, zbuf_v, acc_sh, *sems):
        gsem = sems[:NBUF]
        ssem = sems[NBUF:2 * NBUF]
        sisem = sems[2 * NBUF:3 * NBUF]
        disem = sems[3 * NBUF:4 * NBUF]
        zsem = sems[4 * NBUF]
        c = lax.axis_index("c")
        s = lax.axis_index("s")
        wid = s * NC + c
        row0 = s * ROWS_PER_SUB

        nck = BASE_CK + (wid < EXTRA_TILES)
        cbase = BASE_CK * wid + jnp.minimum(wid, EXTRA_TILES)

        def g_start(b, j):
            pltpu.async_copy(feat_hbm.at[sidx_v.at[b]], rows_v.at[b],
                             gsem[b])

        def g_wait(b):
            pltpu.make_async_copy(feat_hbm.at[pl.ds(0, CH)], rows_v.at[b],
                                  gsem[b]).wait()

        def s_start(b):
            pltpu.async_copy(rows_v.at[b], acc_sh.at[didx_v.at[b]],
                             ssem[b], add=True)

        def s_wait(b):
            pltpu.make_async_copy(feat_hbm.at[pl.ds(0, CH)], rows_v.at[b],
                                  ssem[b]).wait()

        def si_start(b, j):
            pltpu.async_copy(edge_hbm.at[pl.ds((cbase + j) * CH, CH)],
                             sidx_v.at[b], sisem[b])

        def si_wait(b):
            pltpu.make_async_copy(edge_hbm.at[pl.ds(0, CH)], sidx_v.at[b],
                                  sisem[b]).wait()

        def di_start(b, j):
            pltpu.async_copy(edge_hbm.at[pl.ds(E + (cbase + j) * CH, CH)],
                             didx_v.at[b], disem[b])

        def di_wait(b):
            pltpu.make_async_copy(edge_hbm.at[pl.ds(0, CH)], didx_v.at[b],
                                  disem[b]).wait()

        for b in range(NBUF):
            si_start(b, b)
            di_start(b, b)

        z16 = jnp.zeros((16,), jnp.float32)

        @pl.loop(0, ZROWS)
        def _(r):
            for l in range(F // 16):
                zbuf_v[r, pl.ds(16 * l, 16)] = z16

        ZCOP = ROWS_PER_SUB // ZROWS
        ZREM = ROWS_PER_SUB - ZCOP * ZROWS
        for t in range(ZCOP):
            pltpu.async_copy(zbuf_v,
                             acc_sh.at[pl.ds(row0 + t * ZROWS, ZROWS)], zsem)
        pltpu.async_copy(zbuf_v.at[pl.ds(0, ZREM)],
                         acc_sh.at[pl.ds(row0 + ZCOP * ZROWS, ZREM)], zsem)

        for b in range(NBUF):
            si_wait(b)
            g_start(b, b)

        for t in range(ZCOP):
            pltpu.make_async_copy(feat_hbm.at[pl.ds(0, ZROWS)], zbuf_v,
                                  zsem).wait()
        pltpu.make_async_copy(feat_hbm.at[pl.ds(0, ZREM)],
                              zbuf_v.at[pl.ds(0, ZREM)], zsem).wait()
        plsc.subcore_barrier()

        @pl.loop(0, CK_HI, step=NBUF)
        def _(i):
            for b in range(NBUF):
                j = i + b

                @pl.when(j < nck)
                def _():
                    g_wait(b)
                    di_wait(b)
                    s_start(b)

                    @pl.when(j + NBUF < nck)
                    def _():
                        si_start(b, j + NBUF)
            for b in range(NBUF):
                nxt = i + NBUF + b

                @pl.when(nxt < nck)
                def _():
                    s_wait(b)
                    di_start(b, nxt)
                    si_wait(b)
                    g_start(b, nxt)

        for b in range(NBUF):
            s_wait(b)
        plsc.subcore_barrier()

        @pl.when(s < NS - 1)
        def _():
            pltpu.sync_copy(acc_sh.at[pl.ds(row0, ROWS_PER_SUB)],
                            out_hbm.at[c, pl.ds(row0, ROWS_PER_SUB)])

        @pl.when(s == NS - 1)
        def _():
            pltpu.sync_copy(acc_sh.at[pl.ds(row0, ROWS_LAST)],
                            out_hbm.at[c, pl.ds(row0, ROWS_LAST)])

    return k(feat, edge_flat)


BLK = 1000


def _tc_pre_body(feat_ref, w1t_ref, bias_ref, out_ref):
    out_ref[...] = jnp.dot(feat_ref[...], w1t_ref[...],
                           preferred_element_type=jnp.float32,
                           precision=lax.Precision.HIGHEST) + bias_ref[...]


def _tc_pre(feat, w1t, bias):
    return pl.pallas_call(
        _tc_pre_body,
        grid=(N // BLK,),
        in_specs=[
            pl.BlockSpec((BLK, F), lambda i: (i, 0)),
            pl.BlockSpec((F, F), lambda i: (0, 0)),
            pl.BlockSpec((1, F), lambda i: (0, 0)),
        ],
        out_specs=pl.BlockSpec((BLK, F), lambda i: (i, 0)),
        out_shape=jax.ShapeDtypeStruct((N, F), jnp.float32),
    )(feat, w1t, bias)


def _tc_post_body(pre_ref, p0_ref, p1_ref, deg_ref, w2t_ref, out_ref):
    ah = (p0_ref[0] + p1_ref[0]) / deg_ref[...]
    out_ref[...] = pre_ref[...] + jnp.dot(
        ah, w2t_ref[...],
        preferred_element_type=jnp.float32,
        precision=lax.Precision.HIGHEST)


def _tc_post(pre, partials, deg, w2t):
    return pl.pallas_call(
        _tc_post_body,
        grid=(N // BLK,),
        in_specs=[
            pl.BlockSpec((BLK, F), lambda i: (i, 0)),
            pl.BlockSpec((1, BLK, F), lambda i: (0, i, 0)),
            pl.BlockSpec((1, BLK, F), lambda i: (1, i, 0)),
            pl.BlockSpec((BLK, 1), lambda i: (i, 0)),
            pl.BlockSpec((F, F), lambda i: (0, 0)),
        ],
        out_specs=pl.BlockSpec((BLK, F), lambda i: (i, 0)),
        out_shape=jax.ShapeDtypeStruct((N, F), jnp.float32),
    )(pre, partials, partials, deg, w2t)


def kernel(feat, edge_index, in_deg, W1, b1, W2, b2):
    edge_flat = edge_index.reshape(2 * E)
    partials = _sc_segment_sum(feat, edge_flat)

    deg = in_deg.reshape(N, 1)
    bias = (b1 + b2).reshape(1, F)
    pre = _tc_pre(feat, W1.T, bias)
    return _tc_post(pre, partials, deg, W2.T)


# speedup vs baseline: 1.0297x; 1.0297x over previous
import functools

import jax
import jax.numpy as jnp
from jax import lax
from jax.experimental import pallas as pl
from jax.experimental.pallas import tpu as pltpu
from jax.experimental.pallas import tpu_sc as plsc

N = 10000
E = 320000
F = 128

NC = 2
NS = 16
NW = NC * NS
CH = 80
NCHUNKS = E // CH
NBUF = 4
ZROWS = 32
BASE_CK = NCHUNKS // NW
EXTRA_TILES = NCHUNKS - NW * BASE_CK
CK_HI = -(-(BASE_CK + 1) // NBUF) * NBUF
N_ACC = 10112
ROWS_PER_SUB = N_ACC // NS
ROWS_LAST = N - (NS - 1) * ROWS_PER_SUB


def _sc_segment_sum(feat, edge_flat):
    mesh = plsc.VectorSubcoreMesh(core_axis_name="c", subcore_axis_name="s")

    @functools.partial(
        pl.kernel,
        mesh=mesh,
        out_type=jax.ShapeDtypeStruct((NC, N, F), jnp.float32),
        scratch_types=[
            pltpu.VMEM((NBUF, CH), jnp.int32),
            pltpu.VMEM((NBUF, CH), jnp.int32),
            pltpu.VMEM((NBUF, CH, F), jnp.float32),
            pltpu.VMEM((ZROWS, F), jnp.float32),
            pltpu.VMEM_SHARED((N_ACC, F), jnp.float32),
        ] + [pltpu.SemaphoreType.DMA] * (4 * NBUF + 1),
    )
    def k(feat_hbm, edge_hbm, out_hbm,
          sidx_v, didx_v, rows_v, zbuf_v, acc_sh, *sems):
        gsem = sems[:NBUF]
        ssem = sems[NBUF:2 * NBUF]
        sisem = sems[2 * NBUF:3 * NBUF]
        disem = sems[3 * NBUF:4 * NBUF]
        zsem = sems[4 * NBUF]
        c = lax.axis_index("c")
        s = lax.axis_index("s")
        wid = s * NC + c
        row0 = s * ROWS_PER_SUB

        nck = BASE_CK + (wid < EXTRA_TILES)
        cbase = BASE_CK * wid + jnp.minimum(wid, EXTRA_TILES)

        def g_start(b, j):
            pltpu.async_copy(feat_hbm.at[sidx_v.at[b]], rows_v.at[b],
                             gsem[b])

        def g_wait(b):
            pltpu.make_async_copy(feat_hbm.at[pl.ds(0, CH)], rows_v.at[b],
                                  gsem[b]).wait()

        def s_start(b):
            pltpu.async_copy(rows_v.at[b], acc_sh.at[didx_v.at[b]],
                             ssem[b], add=True)

        def s_wait(b):
            pltpu.make_async_copy(feat_hbm.at[pl.ds(0, CH)], rows_v.at[b],
                                  ssem[b]).wait()

        def si_start(b, j):
            pltpu.async_copy(edge_hbm.at[pl.ds((cbase + j) * CH, CH)],
                             sidx_v.at[b], sisem[b])

        def si_wait(b):
            pltpu.make_async_copy(edge_hbm.at[pl.ds(0, CH)], sidx_v.at[b],
                                  sisem[b]).wait()

        def di_start(b, j):
            pltpu.async_copy(edge_hbm.at[pl.ds(E + (cbase + j) * CH, CH)],
                             didx_v.at[b], disem[b])

        def di_wait(b):
            pltpu.make_async_copy(edge_hbm.at[pl.ds(0, CH)], didx_v.at[b],
                                  disem[b]).wait()

        for b in range(NBUF):
            si_start(b, b)
            di_start(b, b)

        z16 = jnp.zeros((16,), jnp.float32)

        @pl.loop(0, ZROWS)
        def _(r):
            for l in range(F // 16):
                zbuf_v[r, pl.ds(16 * l, 16)] = z16

        ZCOP = ROWS_PER_SUB // ZROWS
        ZREM = ROWS_PER_SUB - ZCOP * ZROWS
        for t in range(ZCOP):
            pltpu.async_copy(zbuf_v,
                             acc_sh.at[pl.ds(row0 + t * ZROWS, ZROWS)], zsem)
        pltpu.async_copy(zbuf_v.at[pl.ds(0, ZREM)],
                         acc_sh.at[pl.ds(row0 + ZCOP * ZROWS, ZREM)], zsem)

        for b in range(NBUF):
            si_wait(b)
            g_start(b, b)

        for t in range(ZCOP):
            pltpu.make_async_copy(feat_hbm.at[pl.ds(0, ZROWS)], zbuf_v,
                                  zsem).wait()
        pltpu.make_async_copy(feat_hbm.at[pl.ds(0, ZREM)],
                              zbuf_v.at[pl.ds(0, ZREM)], zsem).wait()
        plsc.subcore_barrier()

        @pl.loop(0, CK_HI, step=NBUF)
        def _(i):
            for b in range(NBUF):
                j = i + b

                @pl.when(j < nck)
                def _():
                    g_wait(b)
                    di_wait(b)
                    s_start(b)

                    @pl.when(j + NBUF < nck)
                    def _():
                        si_start(b, j + NBUF)
            for b in range(NBUF):
                nxt = i + NBUF + b

                @pl.when(nxt < nck)
                def _():
                    s_wait(b)
                    di_start(b, nxt)
                    si_wait(b)
                    g_start(b, nxt)

        for b in range(NBUF):
            s_wait(b)
        plsc.subcore_barrier()

        @pl.when(s < NS - 1)
        def _():
            pltpu.sync_copy(acc_sh.at[pl.ds(row0, ROWS_PER_SUB)],
                            out_hbm.at[c, pl.ds(row0, ROWS_PER_SUB)])

        @pl.when(s == NS - 1)
        def _():
            pltpu.sync_copy(acc_sh.at[pl.ds(row0, ROWS_LAST)],
                            out_hbm.at[c, pl.ds(row0, ROWS_LAST)])

    return k(feat, edge_flat)


BLK = 2000


def _tc_pre_body(feat_ref, w1t_ref, bias_ref, out_ref):
    out_ref[...] = jnp.dot(feat_ref[...], w1t_ref[...],
                           preferred_element_type=jnp.float32,
                           precision=lax.Precision.HIGHEST) + bias_ref[...]


def _tc_pre(feat, w1t, bias):
    return pl.pallas_call(
        _tc_pre_body,
        grid=(N // BLK,),
        in_specs=[
            pl.BlockSpec((BLK, F), lambda i: (i, 0)),
            pl.BlockSpec((F, F), lambda i: (0, 0)),
            pl.BlockSpec((1, F), lambda i: (0, 0)),
        ],
        out_specs=pl.BlockSpec((BLK, F), lambda i: (i, 0)),
        out_shape=jax.ShapeDtypeStruct((N, F), jnp.float32),
    )(feat, w1t, bias)


def _tc_post_body(pre_ref, p0_ref, p1_ref, deg_ref, w2t_ref, out_ref):
    ah = (p0_ref[0] + p1_ref[0]) / deg_ref[...]
    out_ref[...] = pre_ref[...] + jnp.dot(
        ah, w2t_ref[...],
        preferred_element_type=jnp.float32,
        precision=lax.Precision.HIGHEST)


def _tc_post(pre, partials, deg, w2t):
    return pl.pallas_call(
        _tc_post_body,
        grid=(N // BLK,),
        in_specs=[
            pl.BlockSpec((BLK, F), lambda i: (i, 0)),
            pl.BlockSpec((1, BLK, F), lambda i: (0, i, 0)),
            pl.BlockSpec((1, BLK, F), lambda i: (1, i, 0)),
            pl.BlockSpec((BLK, 1), lambda i: (i, 0)),
            pl.BlockSpec((F, F), lambda i: (0, 0)),
        ],
        out_specs=pl.BlockSpec((BLK, F), lambda i: (i, 0)),
        out_shape=jax.ShapeDtypeStruct((N, F), jnp.float32),
    )(pre, partials, partials, deg, w2t)


def kernel(feat, edge_index, in_deg, W1, b1, W2, b2):
    edge_flat = edge_index.reshape(2 * E)
    partials = _sc_segment_sum(feat, edge_flat)

    deg = in_deg.reshape(N, 1)
    bias = (b1 + b2).reshape(1, F)
    pre = _tc_pre(feat, W1.T, bias)
    return _tc_post(pre, partials, deg, W2.T)
